# Initial kernel scaffold; baseline (speedup 1.0000x reference)
import jax
import jax.numpy as jnp
from jax.experimental import pallas as pl


def kernel(x, h, edge_index, edge_attr, W_e1, b_e1, W_e2, b_e2, W_x, W_h1, b_h1, W_h2, b_h2):
    raise NotImplementedError("write your pallas kernel here")


# speedup vs baseline: 6.6443x; 6.6443x over previous
import functools
import jax
import jax.numpy as jnp
from jax import lax
from jax.experimental import pallas as pl
from jax.experimental.pallas import tpu as pltpu
from jax.experimental.pallas import tpu_sc as plsc

DH = 128
DE = 16
DM = 64
P = 128
NC = 2
NS = 16
NW = NC * NS
K = 128

_f32 = jnp.float32


def _pre_body(h_ref, x_ref, wa_ref, wb_ref, a_ref, b_ref):
    h = h_ref[...]
    x = x_ref[...]
    n = h.shape[0]
    pad = jnp.zeros((n, P - DM - 3), _f32)
    a = jnp.dot(h, wa_ref[...], preferred_element_type=_f32)
    b = jnp.dot(h, wb_ref[...], preferred_element_type=_f32)
    a_ref[...] = jnp.concatenate([a, x, pad], axis=1)
    b_ref[...] = jnp.concatenate([b, -x, pad], axis=1)


def _pre(h2, x2, w1a, w1b):
    n = h2.shape[0]
    return pl.pallas_call(
        _pre_body,
        out_shape=[jax.ShapeDtypeStruct((n, P), _f32)] * 2,
    )(h2, x2, w1a, w1b)


def _sc_gather(a_ext, b_ext, src, dst):
    e = src.shape[0]
    nblk = e // K
    bpw = (nblk + NW - 1) // NW
    npair = (bpw + 2) // 2
    mesh = plsc.VectorSubcoreMesh(core_axis_name="c", subcore_axis_name="s")

    @functools.partial(
        pl.kernel,
        out_type=[jax.ShapeDtypeStruct((e, P), _f32)] * 2,
        mesh=mesh,
        scratch_types=[
            pltpu.VMEM((K,), jnp.int32), pltpu.VMEM((K,), jnp.int32),
            pltpu.VMEM((K,), jnp.int32), pltpu.VMEM((K,), jnp.int32),
            pltpu.VMEM((K, P), _f32), pltpu.VMEM((K, P), _f32),
            pltpu.VMEM((K, P), _f32), pltpu.VMEM((K, P), _f32),
            pltpu.SemaphoreType.DMA, pltpu.SemaphoreType.DMA,
            pltpu.SemaphoreType.DMA, pltpu.SemaphoreType.DMA,
        ],
    )
    def k(a_hbm, b_hbm, src_hbm, dst_hbm, ga_hbm, gb_hbm,
          is0, id0, is1, id1, ba0, bb0, ba1, bb1,
          sg0, sg1, so0, so1):
        c = lax.axis_index("c")
        s = lax.axis_index("s")
        wid = s * NC + c
        nv = (nblk - wid + NW - 1) // NW
        idx_s = (is0, is1)
        idx_d = (id0, id1)
        buf_a = (ba0, ba1)
        buf_b = (bb0, bb1)
        sem_g = (sg0, sg1)
        sem_o = (so0, so1)

        def drain_out(b):
            pltpu.make_async_copy(buf_a[b], ga_hbm.at[pl.ds(0, K)],
                                  sem_o[b]).wait()
            pltpu.make_async_copy(buf_b[b], gb_hbm.at[pl.ds(0, K)],
                                  sem_o[b]).wait()

        @pl.loop(0, npair)
        def _(tt):
            for b in (0, 1):
                t = tt * 2 + b
                o = 1 - b

                @pl.when(t < nv)
                def _():
                    base = (t * NW + wid) * K

                    @pl.when(t >= 2)
                    def _():
                        drain_out(b)
                    pltpu.sync_copy(src_hbm.at[pl.ds(base, K)], idx_s[b])
                    pltpu.sync_copy(dst_hbm.at[pl.ds(base, K)], idx_d[b])
                    pltpu.async_copy(a_hbm.at[idx_s[b]], buf_a[b], sem_g[b])
                    pltpu.async_copy(b_hbm.at[idx_d[b]], buf_b[b], sem_g[b])

                @pl.when((t >= 1) & (t - 1 < nv))
                def _():
                    basep = ((t - 1) * NW + wid) * K
                    pltpu.make_async_copy(a_hbm.at[idx_s[o]], buf_a[o],
                                          sem_g[o]).wait()
                    pltpu.make_async_copy(b_hbm.at[idx_d[o]], buf_b[o],
                                          sem_g[o]).wait()
                    pltpu.async_copy(buf_a[o], ga_hbm.at[pl.ds(basep, K)],
                                     sem_o[o])
                    pltpu.async_copy(buf_b[o], gb_hbm.at[pl.ds(basep, K)],
                                     sem_o[o])

        for b in (0, 1):
            @pl.when((nv >= 1) & ((nv - 1) % 2 == b))
            def _():
                drain_out(b)

            @pl.when((nv >= 2) & ((nv - 2) % 2 == b))
            def _():
                drain_out(b)

    return k(a_ext, b_ext, src, dst)


def _edge_body(ga_ref, gb_ref, ea_ref, w1e_ref, b1_ref, wd_ref, w2_ref,
               b2_ref, wx_ref, out_ref):
    ga = ga_ref[...]
    gb = gb_ref[...]
    nb = ga.shape[0]
    pre = ga[:, :DM] + gb[:, :DM]
    diff = ga[:, DM:DM + 3] + gb[:, DM:DM + 3]
    dist2 = jnp.clip(jnp.sum(diff * diff, axis=1, keepdims=True), 1e-12, None)
    t = pre + dist2 * wd_ref[...] + b1_ref[...]
    t = t + jnp.dot(ea_ref[...], w1e_ref[...], preferred_element_type=_f32)
    m1 = jax.nn.silu(t)
    m2 = jax.nn.silu(jnp.dot(m1, w2_ref[...], preferred_element_type=_f32)
                     + b2_ref[...])
    mx = jnp.sum(m2 * wx_ref[...], axis=1, keepdims=True)
    inv = 1.0 / (jnp.sqrt(dist2) + 1e-8)
    dx = mx * inv * diff
    out_ref[...] = jnp.concatenate(
        [m2, dx, jnp.zeros((nb, P - DM - 3), _f32)], axis=1)


def _edge(ga, gb, ea, w1e, b1, wd, w2, b2, wx):
    e = ga.shape[0]
    blk = 2000
    grid = (e // blk,)
    rep = lambda shape: pl.BlockSpec(shape, lambda i: (0, 0))
    return pl.pallas_call(
        _edge_body,
        grid=grid,
        in_specs=[
            pl.BlockSpec((blk, P), lambda i: (i, 0)),
            pl.BlockSpec((blk, P), lambda i: (i, 0)),
            pl.BlockSpec((blk, DE), lambda i: (i, 0)),
            rep((DE, DM)), rep((1, DM)), rep((1, DM)),
            rep((DM, DM)), rep((1, DM)), rep((1, DM)),
        ],
        out_specs=pl.BlockSpec((blk, P), lambda i: (i, 0)),
        out_shape=jax.ShapeDtypeStruct((e, P), _f32),
    )(ga, gb, ea, w1e, b1, wd, w2, b2, wx)


def _sc_scatter(vals, src, zeros_np):
    e = src.shape[0]
    n = zeros_np.shape[0]
    nblk = e // K
    bpw = (nblk + NW - 1) // NW
    npair = (bpw + 2) // 2
    rps = n // NS
    mesh = plsc.VectorSubcoreMesh(core_axis_name="c", subcore_axis_name="s")

    @functools.partial(
        pl.kernel,
        out_type=jax.ShapeDtypeStruct((NC, n, P), _f32),
        mesh=mesh,
        scratch_types=[
            pltpu.VMEM((K,), jnp.int32), pltpu.VMEM((K,), jnp.int32),
            pltpu.VMEM((K, P), _f32), pltpu.VMEM((K, P), _f32),
            pltpu.VMEM_SHARED((n, P), _f32),
            pltpu.SemaphoreType.DMA, pltpu.SemaphoreType.DMA,
        ],
    )
    def k(vals_hbm, src_hbm, z_hbm, out_hbm,
          iv0, iv1, vb0, vb1, acc, sl0, sl1):
        c = lax.axis_index("c")
        s = lax.axis_index("s")
        wid = s * NC + c
        nv = (nblk - wid + NW - 1) // NW
        row0 = s * rps
        idx_v = (iv0, iv1)
        vbuf = (vb0, vb1)
        sem_l = (sl0, sl1)

        pltpu.sync_copy(z_hbm.at[pl.ds(row0, rps)], acc.at[pl.ds(row0, rps)])
        plsc.subcore_barrier()

        @pl.loop(0, npair)
        def _(tt):
            for b in (0, 1):
                t = tt * 2 + b
                o = 1 - b

                @pl.when(t < nv)
                def _():
                    base = (t * NW + wid) * K
                    pltpu.async_copy(src_hbm.at[pl.ds(base, K)], idx_v[b],
                                     sem_l[b])
                    pltpu.async_copy(vals_hbm.at[pl.ds(base, K)], vbuf[b],
                                     sem_l[b])

                @pl.when((t >= 1) & (t - 1 < nv))
                def _():
                    basep = ((t - 1) * NW + wid) * K
                    pltpu.make_async_copy(src_hbm.at[pl.ds(basep, K)],
                                          idx_v[o], sem_l[o]).wait()
                    pltpu.make_async_copy(vals_hbm.at[pl.ds(basep, K)],
                                          vbuf[o], sem_l[o]).wait()
                    pltpu.sync_copy(vbuf[o], acc.at[idx_v[o]], add=True)

        plsc.subcore_barrier()
        pltpu.sync_copy(acc.at[pl.ds(row0, rps)],
                        out_hbm.at[c, pl.ds(row0, rps)])

    return k(vals, src, zeros_np)


def _node_body(h_ref, x_ref, acc_ref, wh1a_ref, wh1b_ref, bh1_ref, wh2_ref,
               bh2_ref, hn_ref, xn_ref):
    h = h_ref[...]
    n = h.shape[0]
    dh = acc_ref[0, :n, :DM] + acc_ref[1, :n, :DM]
    dx = acc_ref[0, :n, DM:DM + 3] + acc_ref[1, :n, DM:DM + 3]
    t = (jnp.dot(h, wh1a_ref[...], preferred_element_type=_f32)
         + jnp.dot(dh, wh1b_ref[...], preferred_element_type=_f32)
         + bh1_ref[...])
    g = jax.nn.silu(t)
    hn_ref[...] = h + jnp.dot(g, wh2_ref[...], preferred_element_type=_f32) \
        + bh2_ref[...]
    xn_ref[...] = x_ref[...] + dx


def _node(h2, x2, acc, wh1a, wh1b, bh1, wh2, bh2):
    n = h2.shape[0]
    return pl.pallas_call(
        _node_body,
        out_shape=[jax.ShapeDtypeStruct((n, DH), _f32),
                   jax.ShapeDtypeStruct((n, 3), _f32)],
    )(h2, x2, acc, wh1a, wh1b, bh1, wh2, bh2)


def kernel(x, h, edge_index, edge_attr,
           W_e1, b_e1, W_e2, b_e2, W_x, W_h1, b_h1, W_h2, b_h2):
    x2 = x[0]
    h2 = h[0]
    ea = edge_attr[0]
    n = h2.shape[0]
    ei = edge_index.astype(jnp.int32).T
    src = ei[0]
    dst = ei[1]

    w1a = W_e1[:DH]
    w1b = W_e1[DH:2 * DH]
    wd = W_e1[2 * DH:2 * DH + 1]
    w1e = W_e1[2 * DH + 1:]

    a_ext, b_ext = _pre(h2, x2, w1a, w1b)
    ga, gb = _sc_gather(a_ext, b_ext, src, dst)
    vals = _edge(ga, gb, ea, w1e, b_e1[None], wd, W_e2, b_e2[None], W_x.T)
    npad = ((n + NW * 8 - 1) // (NW * 8)) * (NW * 8)
    acc = _sc_scatter(vals, src, jnp.zeros((npad, P), _f32))
    hn, xn = _node(h2, x2, acc, W_h1[:DH], W_h1[DH:], b_h1[None],
                   W_h2, b_h2[None])
    return xn[None], hn[None]
